# baseline (speedup 1.0000x reference)
import functools

import jax
import jax.numpy as jnp
from jax import lax
from jax.experimental import pallas as pl
from jax.experimental.pallas import tpu as pltpu
from jax.experimental.pallas import tpu_sc as plsc

BATCH = 16384
DIM = 64
NBUCKETS = 100000
LN_EPS = 1e-5

_NC, _NS = 2, 16
_NW = _NC * _NS
_PLANES_PER = DIM // _NW
_CH = 8192
_UNROLL = 4
_SPIN = 1200


def _sc_gather_t(indices, table_t):
    mesh = plsc.VectorSubcoreMesh(core_axis_name="c", subcore_axis_name="s")

    @functools.partial(
        pl.kernel,
        mesh=mesh,
        out_type=jax.ShapeDtypeStruct((DIM, BATCH), jnp.float32),
        scratch_types=[
            pltpu.VMEM((NBUCKETS,), jnp.float32),
            pltpu.VMEM((_CH,), jnp.int32),
            pltpu.VMEM((_CH,), jnp.float32),
        ],
        compiler_params=pltpu.CompilerParams(
            skip_device_barrier=True, needs_layout_passes=False
        ),
    )
    def gather_kernel(idx_hbm, table_hbm, out_hbm, plane_v, idx_v, orow_v):
        wid = lax.axis_index("s") * _NC + lax.axis_index("c")

        def spin(i, a):
            for _ in range(8):
                a = a * jnp.int32(1664525) + jnp.int32(1013904223)
            return a

        acc = lax.fori_loop(0, (wid & 1) * _SPIN, spin, wid)
        bump = lax.select(acc == jnp.int32(123456789), 1, 0)

        for p in range(_PLANES_PER):
            c = wid * _PLANES_PER + p + bump
            pltpu.sync_copy(table_hbm.at[c], plane_v)
            for ch in range(BATCH // _CH):
                pltpu.sync_copy(idx_hbm.at[pl.ds(ch * _CH, _CH)], idx_v)

                def body(g, _):
                    base = g * 16 * _UNROLL
                    for u in range(_UNROLL):
                        off = base + u * 16
                        iv = idx_v[pl.ds(off, 16)]
                        orow_v[pl.ds(off, 16)] = plsc.load_gather(plane_v, [iv])
                    return 0

                lax.fori_loop(0, _CH // (16 * _UNROLL), body, 0)
                pltpu.sync_copy(orow_v, out_hbm.at[c, pl.ds(ch * _CH, _CH)])

    return gather_kernel(indices, table_t)


def _dense_body_t(e_ref, w_ref, b_ref, g_ref, beta_ref, o_ref):
    h = jnp.dot(w_ref[...], e_ref[...], preferred_element_type=jnp.float32)
    h = h + b_ref[...]
    h = 0.5 * h * (1.0 + lax.erf(h * 0.7071067811865476))
    mean = jnp.mean(h, axis=0, keepdims=True)
    c = h - mean
    var = jnp.mean(c * c, axis=0, keepdims=True)
    o_ref[...] = c * lax.rsqrt(var + LN_EPS) * g_ref[...] + beta_ref[...]


def _tc_dense_t(emb_t, W, b2, g2, be2):
    blk = 8192
    grid = (BATCH // blk,)
    return pl.pallas_call(
        _dense_body_t,
        grid=grid,
        in_specs=[
            pl.BlockSpec((DIM, blk), lambda i: (0, i)),
            pl.BlockSpec((DIM, DIM), lambda i: (0, 0)),
            pl.BlockSpec((DIM, 1), lambda i: (0, 0)),
            pl.BlockSpec((DIM, 1), lambda i: (0, 0)),
            pl.BlockSpec((DIM, 1), lambda i: (0, 0)),
        ],
        out_specs=pl.BlockSpec((DIM, blk), lambda i: (0, i)),
        out_shape=jax.ShapeDtypeStruct((DIM, BATCH), jnp.float32),
    )(emb_t, W, b2, g2, be2)


def kernel(indices, table, W, b, gamma, beta):
    emb_t = _sc_gather_t(indices.astype(jnp.int32), table.T)
    out_t = _tc_dense_t(
        emb_t,
        W,
        b.reshape(DIM, 1),
        gamma.reshape(DIM, 1),
        beta.reshape(DIM, 1),
    )
    return out_t.T


# speedup vs baseline: 1.0707x; 1.0707x over previous
import functools

import jax
import jax.numpy as jnp
from jax import lax
from jax.experimental import pallas as pl
from jax.experimental.pallas import tpu as pltpu
from jax.experimental.pallas import tpu_sc as plsc

BATCH = 16384
DIM = 64
NBUCKETS = 100000
LN_EPS = 1e-5

_NC, _NS = 2, 16
_NW = _NC * _NS
_PLANES_PER = DIM // _NW
_CH = 8192
_UNROLL = 4


def _sc_gather_t(indices, table_t):
    mesh = plsc.VectorSubcoreMesh(core_axis_name="c", subcore_axis_name="s")

    @functools.partial(
        pl.kernel,
        mesh=mesh,
        out_type=jax.ShapeDtypeStruct((DIM, BATCH), jnp.float32),
        scratch_types=[
            pltpu.VMEM((NBUCKETS,), jnp.float32),
            pltpu.VMEM((_CH,), jnp.int32),
            pltpu.VMEM((_CH,), jnp.float32),
        ],
        compiler_params=pltpu.CompilerParams(
            skip_device_barrier=True, needs_layout_passes=False
        ),
    )
    def gather_kernel(idx_hbm, table_hbm, out_hbm, plane_v, idx_v, orow_v):
        wid = lax.axis_index("s") * _NC + lax.axis_index("c")
        for p in range(_PLANES_PER):
            c = wid * _PLANES_PER + p
            pltpu.sync_copy(table_hbm.at[c], plane_v)
            for ch in range(BATCH // _CH):
                pltpu.sync_copy(idx_hbm.at[pl.ds(ch * _CH, _CH)], idx_v)

                def body(g, _):
                    base = g * 16 * _UNROLL
                    for u in range(_UNROLL):
                        off = base + u * 16
                        iv = idx_v[pl.ds(off, 16)]
                        orow_v[pl.ds(off, 16)] = plsc.load_gather(plane_v, [iv])
                    return 0

                lax.fori_loop(0, _CH // (16 * _UNROLL), body, 0)
                pltpu.sync_copy(orow_v, out_hbm.at[c, pl.ds(ch * _CH, _CH)])

    return gather_kernel(indices, table_t)


def _dense_body_t(e_ref, w_ref, b_ref, g_ref, beta_ref, o_ref):
    h = jnp.dot(w_ref[...], e_ref[...], preferred_element_type=jnp.float32)
    h = h + b_ref[...]
    h = 0.5 * h * (1.0 + lax.erf(h * 0.7071067811865476))
    mean = jnp.mean(h, axis=0, keepdims=True)
    c = h - mean
    var = jnp.mean(c * c, axis=0, keepdims=True)
    o_ref[...] = c * lax.rsqrt(var + LN_EPS) * g_ref[...] + beta_ref[...]


def _tc_dense_t(emb_t, W, b2, g2, be2):
    blk = 8192
    grid = (BATCH // blk,)
    return pl.pallas_call(
        _dense_body_t,
        grid=grid,
        in_specs=[
            pl.BlockSpec((DIM, blk), lambda i: (0, i)),
            pl.BlockSpec((DIM, DIM), lambda i: (0, 0)),
            pl.BlockSpec((DIM, 1), lambda i: (0, 0)),
            pl.BlockSpec((DIM, 1), lambda i: (0, 0)),
            pl.BlockSpec((DIM, 1), lambda i: (0, 0)),
        ],
        out_specs=pl.BlockSpec((DIM, blk), lambda i: (0, i)),
        out_shape=jax.ShapeDtypeStruct((DIM, BATCH), jnp.float32),
    )(emb_t, W, b2, g2, be2)


def kernel(indices, table, W, b, gamma, beta):
    emb_t = _sc_gather_t(indices.astype(jnp.int32), table.T)
    out_t = _tc_dense_t(
        emb_t,
        W,
        b.reshape(DIM, 1),
        gamma.reshape(DIM, 1),
        beta.reshape(DIM, 1),
    )
    return out_t.T
